# baseline (speedup 1.0000x reference)
import functools

import jax
import jax.numpy as jnp
from jax import lax
from jax.experimental import pallas as pl
from jax.experimental.pallas import tpu as pltpu
from jax.experimental.pallas import tpu_sc as plsc

N_NODES = 50000
N_EDGES = 800000
F = 4
L = 16
S = 5
NR = 8
H = 16

N_PAD = 50176
E_PAD = 819200
EB = 4096
NB = 1024
K = 128
KS = 128
DEPTH = 4
N_TILES = 16
TILE_E = E_PAD // N_TILES
N_BLKS = TILE_E // K
NI = N_BLKS // DEPTH
TILE_N = N_PAD // N_TILES
FH = F // 2
CW = FH * L


def _edge_body(vt_ref, eps_ref, wr1a_ref, wr2a_ref, wr1b_ref, wr2b_ref,
               y_ref, w0_ref, w1_ref):
    x = vt_ref[0:1, :]
    y = vt_ref[1:2, :]
    z = vt_ref[2:3, :]
    r2 = x * x + y * y + z * z + 1e-12
    inv = lax.rsqrt(r2)
    r = r2 * inv
    ux = x * inv
    uy = y * inv
    uz = z * inv
    Y = jnp.concatenate([
        jnp.ones_like(ux),
        ux, uy, uz,
        ux * uy, uy * uz, 3.0 * uz * uz - 1.0, ux * uz, ux * ux - uy * uy,
        uy * (3.0 * ux * ux - uy * uy), ux * uy * uz,
        uy * (5.0 * uz * uz - 1.0), uz * (5.0 * uz * uz - 3.0),
        ux * (5.0 * uz * uz - 1.0), uz * (ux * ux - uy * uy),
        ux * (ux * ux - 3.0 * uy * uy),
    ], axis=0)
    y_ref[...] = Y
    u = r
    u2 = u * u
    u3 = u2 * u
    u6 = u3 * u3
    u7 = u6 * u
    u8 = u6 * u2
    env = 1.0 - 28.0 * u6 + 48.0 * u7 - 21.0 * u8
    env = jnp.where(u < 1.0, env, 0.0)
    s1 = jnp.sin(jnp.pi * u)
    c1 = jnp.cos(jnp.pi * u)
    two_c1 = 2.0 * c1
    sins = [s1, two_c1 * s1]
    for _ in range(NR - 2):
        sins.append(two_c1 * sins[-1] - sins[-2])
    scale = jnp.sqrt(jnp.float32(2.0)) * env / (u + 1e-9)
    rbf = jnp.concatenate([sn * scale for sn in sins], axis=0)
    eps = eps_ref[0:1, 0:1]
    for wr1t, wr2t, wref in ((wr1a_ref, wr2a_ref, w0_ref),
                             (wr1b_ref, wr2b_ref, w1_ref)):
        t = jnp.dot(wr1t[...], rbf, preferred_element_type=jnp.float32)
        t = t * jax.nn.sigmoid(t)
        w = jnp.dot(wr2t[...], t, preferred_element_type=jnp.float32) * eps
        wref[...] = w


def _edge_precompute(vt, eps11, wr1t_0, wr2t_0, wr1t_1, wr2t_1):
    return pl.pallas_call(
        _edge_body,
        grid=(E_PAD // EB,),
        in_specs=[
            pl.BlockSpec((3, EB), lambda i: (0, i)),
            pl.BlockSpec((1, 1), lambda i: (0, 0)),
            pl.BlockSpec((32, NR), lambda i: (0, 0)),
            pl.BlockSpec((F, 32), lambda i: (0, 0)),
            pl.BlockSpec((32, NR), lambda i: (0, 0)),
            pl.BlockSpec((F, 32), lambda i: (0, 0)),
        ],
        out_specs=[
            pl.BlockSpec((L, EB), lambda i: (0, i)),
            pl.BlockSpec((F, EB), lambda i: (0, i)),
            pl.BlockSpec((F, EB), lambda i: (0, i)),
        ],
        out_shape=[
            jax.ShapeDtypeStruct((L, E_PAD), jnp.float32),
            jax.ShapeDtypeStruct((F, E_PAD), jnp.float32),
            jax.ShapeDtypeStruct((F, E_PAD), jnp.float32),
        ],
    )(vt, eps11, wr1t_0, wr2t_0, wr1t_1, wr2t_1)


def _init_body(sp_ref, t0_ref, out_ref):
    sp = sp_ref[...]
    oh = (sp == lax.broadcasted_iota(jnp.int32, (1, S), 1)).astype(jnp.float32)
    f0 = jnp.dot(oh, t0_ref[...], preferred_element_type=jnp.float32)
    out_ref[0] = f0[:, :CW]
    out_ref[1] = f0[:, CW:]


def _init_feats(sp, t0):
    return pl.pallas_call(
        _init_body,
        grid=(N_PAD // NB,),
        in_specs=[
            pl.BlockSpec((NB, 1), lambda i: (i, 0)),
            pl.BlockSpec((S, F * L), lambda i: (0, 0)),
        ],
        out_specs=pl.BlockSpec((2, NB, CW), lambda i: (0, i, 0)),
        out_shape=jax.ShapeDtypeStruct((2, N_PAD, CW), jnp.float32),
    )(sp, t0)


def _onehot(sp_ref):
    sp = sp_ref[...]
    return (sp == lax.broadcasted_iota(jnp.int32, (1, S), 1)).astype(jnp.float32)


def _product(h2, oh, wp_ref, mmix_ref):
    wp = jnp.dot(oh, wp_ref[...], preferred_element_type=jnp.float32)
    cols = []
    for f in range(F):
        hf = h2[:, f * L:(f + 1) * L]
        nrm = jnp.sum(hf * hf, axis=1, keepdims=True)
        scale = (wp[:, 3 * f:3 * f + 1] + wp[:, 3 * f + 1:3 * f + 2] * nrm
                 + wp[:, 3 * f + 2:3 * f + 3] * nrm * nrm)
        cols.append(hf * scale)
    hs = jnp.concatenate(cols, axis=1)
    return jnp.dot(hs, mmix_ref[...], preferred_element_type=jnp.float32)


def _species_matmul(h, oh, m_ref):
    wsel = jnp.dot(oh, m_ref[...], preferred_element_type=jnp.float32)
    outs = []
    for g in range(F):
        acc = h[:, :L] * wsel[:, g:g + 1]
        for f in range(1, F):
            acc = acc + h[:, f * L:(f + 1) * L] * wsel[:, f * F + g:f * F + g + 1]
        outs.append(acc)
    return jnp.concatenate(outs, axis=1)


def _node1_body(agg_ref, sp_ref, msf_ref, wp_ref, mmix_ref, out_ref):
    h = jnp.concatenate([agg_ref[0], agg_ref[1]], axis=1)
    oh = _onehot(sp_ref)
    h2 = _species_matmul(h, oh, msf_ref)
    feats = _product(h2, oh, wp_ref, mmix_ref)
    out_ref[0] = feats[:, :CW]
    out_ref[1] = feats[:, CW:]


def _node1(agg, sp, msf, wp0, mmix0):
    return pl.pallas_call(
        _node1_body,
        grid=(N_PAD // NB,),
        in_specs=[
            pl.BlockSpec((2, NB, CW), lambda i: (0, i, 0)),
            pl.BlockSpec((NB, 1), lambda i: (i, 0)),
            pl.BlockSpec((S, F * F), lambda i: (0, 0)),
            pl.BlockSpec((S, F * 3), lambda i: (0, 0)),
            pl.BlockSpec((F * L, F * L), lambda i: (0, 0)),
        ],
        out_specs=pl.BlockSpec((2, NB, CW), lambda i: (0, i, 0)),
        out_shape=jax.ShapeDtypeStruct((2, N_PAD, CW), jnp.float32),
    )(agg, sp, msf, wp0, mmix0)


def _node2_body(agg_ref, fts_ref, sp_ref, mskip_ref, wp_ref, mmix_ref,
                k1_ref, wm2_ref, out_ref):
    h = jnp.concatenate([agg_ref[0], agg_ref[1]], axis=1)
    fts = jnp.concatenate([fts_ref[0], fts_ref[1]], axis=1)
    oh = _onehot(sp_ref)
    sc = _species_matmul(fts, oh, mskip_ref)
    feats2 = _product(h, oh, wp_ref, mmix_ref) + sc
    t = jnp.dot(feats2, k1_ref[...], preferred_element_type=jnp.float32)
    t = t * jax.nn.sigmoid(t)
    out_ref[...] = jnp.dot(t, wm2_ref[...], preferred_element_type=jnp.float32)


def _node2(agg, fts, sp, mskip, wp1, mmix1, k1, wm2):
    return pl.pallas_call(
        _node2_body,
        grid=(N_PAD // NB,),
        in_specs=[
            pl.BlockSpec((2, NB, CW), lambda i: (0, i, 0)),
            pl.BlockSpec((2, NB, CW), lambda i: (0, i, 0)),
            pl.BlockSpec((NB, 1), lambda i: (i, 0)),
            pl.BlockSpec((S, F * F), lambda i: (0, 0)),
            pl.BlockSpec((S, F * 3), lambda i: (0, 0)),
            pl.BlockSpec((F * L, F * L), lambda i: (0, 0)),
            pl.BlockSpec((F * L, H), lambda i: (0, 0)),
            pl.BlockSpec((H, 1), lambda i: (0, 0)),
        ],
        out_specs=pl.BlockSpec((NB, 1), lambda i: (i, 0)),
        out_shape=jax.ShapeDtypeStruct((N_PAD, 1), jnp.float32),
    )(agg, fts, sp, mskip, wp1, mmix1, k1, wm2)


def _sc_body(tbl, wt, snd, rcv2d, yt, zeros, out,
             snd_v, rcv_v, ridx_v, idx_v, y_v, w_v, sf_v, acc,
             lsem, gsem, ssem):
    c = lax.axis_index("c")
    s = lax.axis_index("s")
    pltpu.sync_copy(zeros, acc.at[pl.ds(s * TILE_N, TILE_N)])
    plsc.subcore_barrier()
    tile_base = s * TILE_E
    row_base = tile_base // KS
    cN = c * N_PAD

    def lin_copies(d, b):
        base = tile_base + b * K
        return (
            (snd.at[pl.ds(base, K)], snd_v.at[d]),
            (rcv2d.at[row_base + b], rcv_v.at[d]),
            (yt.at[:, pl.ds(base, K)], y_v.at[d]),
            (wt.at[pl.ds(2 * c, FH), pl.ds(base, K)], w_v.at[d]),
        )

    def issue_lin(d, b):
        for src, dst in lin_copies(d, b):
            pltpu.async_copy(src, dst, lsem.at[d])

    def drain_lin(d, b):
        for src, dst in lin_copies(d, b):
            pltpu.make_async_copy(src, dst, lsem.at[d]).wait()

    def copy_ridx_idx_issue_gather(d):
        for k in range(K // 16):
            sl = pl.ds(k * 16, 16)
            ridx_v[d, sl] = rcv_v[d, sl]
            idx_v[d, sl] = snd_v[d, sl] + cN
        pltpu.async_copy(tbl.at[idx_v.at[d]], sf_v.at[d], gsem.at[d])

    def drain_gather(d):
        pltpu.make_async_copy(tbl.at[idx_v.at[d]], sf_v.at[d],
                              gsem.at[d]).wait()

    def issue_scatter(d):
        pltpu.async_copy(sf_v.at[d], acc.at[ridx_v.at[d]], ssem.at[d],
                         add=True)

    def drain_scatter(d):
        pltpu.make_async_copy(sf_v.at[d], acc.at[ridx_v.at[d]],
                              ssem.at[d]).wait()

    def compute(d):
        @functools.partial(plsc.parallel_loop, 0, K // 16, unroll=2)
        def grp_body(g):
            eb = g * 16
            rows = jnp.arange(16, dtype=jnp.int32) + eb
            ys = [y_v[d, l, pl.ds(eb, 16)] for l in range(L)]
            for ff in range(FH):
                wv = w_v[d, ff, pl.ds(eb, 16)]
                cols = [jnp.full((16,), ff * L + l, jnp.int32)
                        for l in range(L)]
                sfs = [plsc.load_gather(sf_v.at[d], [rows, cols[l]])
                       for l in range(L)]
                sa = sfs[0] * ys[0]
                sb = sfs[1] * ys[1]
                for l in range(2, L, 2):
                    sa = sa + sfs[l] * ys[l]
                    sb = sb + sfs[l + 1] * ys[l + 1]
                sdot = sa + sb
                for l in range(L):
                    m = wv * (sfs[l] + sdot * ys[l])
                    plsc.store_scatter(sf_v.at[d], [rows, cols[l]], m)

    for d in range(DEPTH):
        issue_lin(d, d)

    def it_body(i, carry):
        b0 = DEPTH * i
        for d in range(DEPTH):
            @pl.when(i > 0)
            def _(d=d):
                drain_scatter(d)

            drain_lin(d, b0 + d)
            copy_ridx_idx_issue_gather(d)
        for d in range(DEPTH):
            drain_gather(d)
            compute(d)
            issue_scatter(d)

            @pl.when(i < NI - 1)
            def _(d=d, b=b0 + d):
                issue_lin(d, b + DEPTH)

        return carry

    lax.fori_loop(0, NI, it_body, 0)
    for d in range(DEPTH):
        drain_scatter(d)
    plsc.subcore_barrier()
    pltpu.sync_copy(acc.at[pl.ds(s * TILE_N, TILE_N)],
                    out.at[pl.ds(cN + s * TILE_N, TILE_N)])


_sc_interact = functools.partial(
    pl.kernel,
    out_type=jax.ShapeDtypeStruct((2 * N_PAD, CW), jnp.float32),
    mesh=plsc.VectorSubcoreMesh(core_axis_name="c", subcore_axis_name="s"),
    scratch_types=[
        pltpu.VMEM((DEPTH, K), jnp.int32),
        pltpu.VMEM((DEPTH, K), jnp.int32),
        pltpu.VMEM((DEPTH, K), jnp.int32),
        pltpu.VMEM((DEPTH, K), jnp.int32),
        pltpu.VMEM((DEPTH, L, K), jnp.float32),
        pltpu.VMEM((DEPTH, FH, K), jnp.float32),
        pltpu.VMEM((DEPTH, K, CW), jnp.float32),
        pltpu.VMEM_SHARED((N_PAD, CW), jnp.float32),
        pltpu.SemaphoreType.DMA((DEPTH,)),
        pltpu.SemaphoreType.DMA((DEPTH,)),
        pltpu.SemaphoreType.DMA((DEPTH,)),
    ],
    compiler_params=pltpu.CompilerParams(use_tc_tiling_on_sc=False),
)(_sc_body)


def kernel(vectors, senders, receivers, node_species, varepsilon, embed,
           Wr1_0, Wr2_0, Wmix_0, Wprod_0, Wskipfirst_0,
           Wr1_1, Wr2_1, Wmix_1, Wprod_1, Wskip_1, Wm1, Wm2):
    f32 = jnp.float32
    pad_e = E_PAD - N_EDGES
    vt = jnp.concatenate(
        [vectors.astype(f32).T,
         jnp.concatenate([jnp.ones((1, pad_e), f32),
                          jnp.zeros((2, pad_e), f32)], axis=0)], axis=1)
    snd = jnp.concatenate(
        [senders.astype(jnp.int32), jnp.zeros((pad_e,), jnp.int32)])
    rcv = jnp.concatenate(
        [receivers.astype(jnp.int32),
         jnp.full((pad_e,), N_NODES, jnp.int32)])
    sp = jnp.concatenate(
        [node_species.astype(jnp.int32),
         jnp.zeros((N_PAD - N_NODES,), jnp.int32)])[:, None]
    eps11 = (1.0 / jnp.sqrt(1.0 + varepsilon ** 2)).reshape(1, 1).astype(f32)
    eye = jnp.eye(L, dtype=f32)
    t0 = jnp.zeros((S, F * L), f32).at[:, jnp.arange(F) * L].set(embed)
    msf = Wskipfirst_0.reshape(S, F * F)
    mskip = Wskip_1.reshape(S, F * F)
    mmix0 = jnp.kron(Wmix_0, eye)
    mmix1 = jnp.kron(Wmix_1, eye)
    wp0 = Wprod_0.reshape(S, F * 3)
    wp1 = Wprod_1.reshape(S, F * 3)
    k1 = jnp.zeros((F * L, H), f32).at[jnp.arange(F) * L].set(Wm1)
    zeros_tile = jnp.zeros((TILE_N, CW), f32)

    Yt, w0t, w1t = _edge_precompute(vt, eps11, Wr1_0.T, Wr2_0.T, Wr1_1.T,
                                    Wr2_1.T)

    rcv2d = rcv.reshape(E_PAD // KS, KS)
    f0 = _init_feats(sp, t0)
    a0 = _sc_interact(f0.reshape(2 * N_PAD, CW), w0t, snd, rcv2d, Yt,
                      zeros_tile)
    f1 = _node1(a0.reshape(2, N_PAD, CW), sp, msf, wp0, mmix0)
    a1 = _sc_interact(f1.reshape(2 * N_PAD, CW), w1t, snd, rcv2d, Yt,
                      zeros_tile)
    out = _node2(a1.reshape(2, N_PAD, CW), f1, sp, mskip, wp1, mmix1, k1, Wm2)
    return out[:N_NODES]


# speedup vs baseline: 1.1451x; 1.1451x over previous
import functools

import jax
import jax.numpy as jnp
from jax import lax
from jax.experimental import pallas as pl
from jax.experimental.pallas import tpu as pltpu
from jax.experimental.pallas import tpu_sc as plsc

N_NODES = 50000
N_EDGES = 800000
F = 4
L = 16
S = 5
NR = 8
H = 16

N_PAD = 50176
E_PAD = 819200
EB = 4096
NB = 1024
K = 128
KS = 128
DEPTH = 4
N_TILES = 16
TILE_E = E_PAD // N_TILES
N_BLKS = TILE_E // K
NI = N_BLKS // DEPTH
TILE_N = N_PAD // N_TILES
FH = F // 2
CW = FH * L


def _edge_body(vt_ref, eps_ref, wr1a_ref, wr2a_ref, wr1b_ref, wr2b_ref,
               y_ref, w0_ref, w1_ref):
    x = vt_ref[0:1, :]
    y = vt_ref[1:2, :]
    z = vt_ref[2:3, :]
    r2 = x * x + y * y + z * z + 1e-12
    inv = lax.rsqrt(r2)
    r = r2 * inv
    ux = x * inv
    uy = y * inv
    uz = z * inv
    Y = jnp.concatenate([
        jnp.ones_like(ux),
        ux, uy, uz,
        ux * uy, uy * uz, 3.0 * uz * uz - 1.0, ux * uz, ux * ux - uy * uy,
        uy * (3.0 * ux * ux - uy * uy), ux * uy * uz,
        uy * (5.0 * uz * uz - 1.0), uz * (5.0 * uz * uz - 3.0),
        ux * (5.0 * uz * uz - 1.0), uz * (ux * ux - uy * uy),
        ux * (ux * ux - 3.0 * uy * uy),
    ], axis=0)
    y_ref[...] = Y
    u = r
    u2 = u * u
    u3 = u2 * u
    u6 = u3 * u3
    u7 = u6 * u
    u8 = u6 * u2
    env = 1.0 - 28.0 * u6 + 48.0 * u7 - 21.0 * u8
    env = jnp.where(u < 1.0, env, 0.0)
    s1 = jnp.sin(jnp.pi * u)
    c1 = jnp.cos(jnp.pi * u)
    two_c1 = 2.0 * c1
    sins = [s1, two_c1 * s1]
    for _ in range(NR - 2):
        sins.append(two_c1 * sins[-1] - sins[-2])
    scale = jnp.sqrt(jnp.float32(2.0)) * env / (u + 1e-9)
    rbf = jnp.concatenate([sn * scale for sn in sins], axis=0)
    eps = eps_ref[0:1, 0:1]
    for wr1t, wr2t, wref in ((wr1a_ref, wr2a_ref, w0_ref),
                             (wr1b_ref, wr2b_ref, w1_ref)):
        t = jnp.dot(wr1t[...], rbf, preferred_element_type=jnp.float32)
        t = t * jax.nn.sigmoid(t)
        w = jnp.dot(wr2t[...], t, preferred_element_type=jnp.float32) * eps
        wref[...] = w


def _edge_precompute(vt, eps11, wr1t_0, wr2t_0, wr1t_1, wr2t_1):
    return pl.pallas_call(
        _edge_body,
        grid=(E_PAD // EB,),
        in_specs=[
            pl.BlockSpec((3, EB), lambda i: (0, i)),
            pl.BlockSpec((1, 1), lambda i: (0, 0)),
            pl.BlockSpec((32, NR), lambda i: (0, 0)),
            pl.BlockSpec((F, 32), lambda i: (0, 0)),
            pl.BlockSpec((32, NR), lambda i: (0, 0)),
            pl.BlockSpec((F, 32), lambda i: (0, 0)),
        ],
        out_specs=[
            pl.BlockSpec((L, EB), lambda i: (0, i)),
            pl.BlockSpec((F, EB), lambda i: (0, i)),
            pl.BlockSpec((F, EB), lambda i: (0, i)),
        ],
        out_shape=[
            jax.ShapeDtypeStruct((L, E_PAD), jnp.float32),
            jax.ShapeDtypeStruct((F, E_PAD), jnp.float32),
            jax.ShapeDtypeStruct((F, E_PAD), jnp.float32),
        ],
    )(vt, eps11, wr1t_0, wr2t_0, wr1t_1, wr2t_1)


def _init_body(sp_ref, t0_ref, out_ref):
    sp = sp_ref[...]
    oh = (sp == lax.broadcasted_iota(jnp.int32, (1, S), 1)).astype(jnp.float32)
    f0 = jnp.dot(oh, t0_ref[...], preferred_element_type=jnp.float32)
    out_ref[0] = f0[:, :CW]
    out_ref[1] = f0[:, CW:]


def _init_feats(sp, t0):
    return pl.pallas_call(
        _init_body,
        grid=(N_PAD // NB,),
        in_specs=[
            pl.BlockSpec((NB, 1), lambda i: (i, 0)),
            pl.BlockSpec((S, F * L), lambda i: (0, 0)),
        ],
        out_specs=pl.BlockSpec((2, NB, CW), lambda i: (0, i, 0)),
        out_shape=jax.ShapeDtypeStruct((2, N_PAD, CW), jnp.float32),
    )(sp, t0)


def _onehot(sp_ref):
    sp = sp_ref[...]
    return (sp == lax.broadcasted_iota(jnp.int32, (1, S), 1)).astype(jnp.float32)


def _product(h2, oh, wp_ref, mmix_ref):
    wp = jnp.dot(oh, wp_ref[...], preferred_element_type=jnp.float32)
    cols = []
    for f in range(F):
        hf = h2[:, f * L:(f + 1) * L]
        nrm = jnp.sum(hf * hf, axis=1, keepdims=True)
        scale = (wp[:, 3 * f:3 * f + 1] + wp[:, 3 * f + 1:3 * f + 2] * nrm
                 + wp[:, 3 * f + 2:3 * f + 3] * nrm * nrm)
        cols.append(hf * scale)
    hs = jnp.concatenate(cols, axis=1)
    return jnp.dot(hs, mmix_ref[...], preferred_element_type=jnp.float32)


def _species_matmul(h, oh, m_ref):
    A = jnp.concatenate([oh[:, s:s + 1] * h for s in range(S)], axis=1)
    return jnp.dot(A, m_ref[...], preferred_element_type=jnp.float32)


def _node1_body(agg_ref, sp_ref, msf_ref, wp_ref, mmix_ref, out_ref):
    h = jnp.concatenate([agg_ref[0], agg_ref[1]], axis=1)
    oh = _onehot(sp_ref)
    h2 = _species_matmul(h, oh, msf_ref)
    feats = _product(h2, oh, wp_ref, mmix_ref)
    out_ref[0] = feats[:, :CW]
    out_ref[1] = feats[:, CW:]


def _node1(agg, sp, msf, wp0, mmix0):
    return pl.pallas_call(
        _node1_body,
        grid=(N_PAD // NB,),
        in_specs=[
            pl.BlockSpec((2, NB, CW), lambda i: (0, i, 0)),
            pl.BlockSpec((NB, 1), lambda i: (i, 0)),
            pl.BlockSpec((S * F * L, F * L), lambda i: (0, 0)),
            pl.BlockSpec((S, F * 3), lambda i: (0, 0)),
            pl.BlockSpec((F * L, F * L), lambda i: (0, 0)),
        ],
        out_specs=pl.BlockSpec((2, NB, CW), lambda i: (0, i, 0)),
        out_shape=jax.ShapeDtypeStruct((2, N_PAD, CW), jnp.float32),
    )(agg, sp, msf, wp0, mmix0)


def _node2_body(agg_ref, fts_ref, sp_ref, mskip_ref, wp_ref, mmix_ref,
                k1_ref, wm2_ref, out_ref):
    h = jnp.concatenate([agg_ref[0], agg_ref[1]], axis=1)
    fts = jnp.concatenate([fts_ref[0], fts_ref[1]], axis=1)
    oh = _onehot(sp_ref)
    sc = _species_matmul(fts, oh, mskip_ref)
    feats2 = _product(h, oh, wp_ref, mmix_ref) + sc
    t = jnp.dot(feats2, k1_ref[...], preferred_element_type=jnp.float32)
    t = t * jax.nn.sigmoid(t)
    out_ref[...] = jnp.dot(t, wm2_ref[...], preferred_element_type=jnp.float32)


def _node2(agg, fts, sp, mskip, wp1, mmix1, k1, wm2):
    return pl.pallas_call(
        _node2_body,
        grid=(N_PAD // NB,),
        in_specs=[
            pl.BlockSpec((2, NB, CW), lambda i: (0, i, 0)),
            pl.BlockSpec((2, NB, CW), lambda i: (0, i, 0)),
            pl.BlockSpec((NB, 1), lambda i: (i, 0)),
            pl.BlockSpec((S * F * L, F * L), lambda i: (0, 0)),
            pl.BlockSpec((S, F * 3), lambda i: (0, 0)),
            pl.BlockSpec((F * L, F * L), lambda i: (0, 0)),
            pl.BlockSpec((F * L, H), lambda i: (0, 0)),
            pl.BlockSpec((H, 1), lambda i: (0, 0)),
        ],
        out_specs=pl.BlockSpec((NB, 1), lambda i: (i, 0)),
        out_shape=jax.ShapeDtypeStruct((N_PAD, 1), jnp.float32),
    )(agg, fts, sp, mskip, wp1, mmix1, k1, wm2)


def _sc_body(tbl, wt, snd, rcv2d, yt, zeros, out,
             snd_v, rcv_v, ridx_v, idx_v, y_v, w_v, sf_v, acc,
             lsem, gsem, ssem):
    c = lax.axis_index("c")
    s = lax.axis_index("s")
    pltpu.sync_copy(zeros, acc.at[pl.ds(s * TILE_N, TILE_N)])
    plsc.subcore_barrier()
    tile_base = s * TILE_E
    row_base = tile_base // KS
    cN = c * N_PAD

    def lin_copies(d, b):
        base = tile_base + b * K
        return (
            (snd.at[pl.ds(base, K)], snd_v.at[d]),
            (rcv2d.at[row_base + b], rcv_v.at[d]),
            (yt.at[:, pl.ds(base, K)], y_v.at[d]),
            (wt.at[pl.ds(2 * c, FH), pl.ds(base, K)], w_v.at[d]),
        )

    def issue_lin(d, b):
        for src, dst in lin_copies(d, b):
            pltpu.async_copy(src, dst, lsem.at[d])

    def drain_lin(d, b):
        for src, dst in lin_copies(d, b):
            pltpu.make_async_copy(src, dst, lsem.at[d]).wait()

    def copy_ridx_idx_issue_gather(d):
        for k in range(K // 16):
            sl = pl.ds(k * 16, 16)
            ridx_v[d, sl] = rcv_v[d, sl]
            idx_v[d, sl] = snd_v[d, sl] + cN
        pltpu.async_copy(tbl.at[idx_v.at[d]], sf_v.at[d], gsem.at[d])

    def drain_gather(d):
        pltpu.make_async_copy(tbl.at[idx_v.at[d]], sf_v.at[d],
                              gsem.at[d]).wait()

    def issue_scatter(d):
        pltpu.async_copy(sf_v.at[d], acc.at[ridx_v.at[d]], ssem.at[d],
                         add=True)

    def drain_scatter(d):
        pltpu.make_async_copy(sf_v.at[d], acc.at[ridx_v.at[d]],
                              ssem.at[d]).wait()

    def compute(d):
        @functools.partial(plsc.parallel_loop, 0, K // 16, unroll=2)
        def grp_body(g):
            eb = g * 16
            rows = jnp.arange(16, dtype=jnp.int32) + eb
            ys = [y_v[d, l, pl.ds(eb, 16)] for l in range(L)]
            for ff in range(FH):
                wv = w_v[d, ff, pl.ds(eb, 16)]
                cols = [jnp.full((16,), ff * L + l, jnp.int32)
                        for l in range(L)]
                sfs = [plsc.load_gather(sf_v.at[d], [rows, cols[l]])
                       for l in range(L)]
                sa = sfs[0] * ys[0]
                sb = sfs[1] * ys[1]
                for l in range(2, L, 2):
                    sa = sa + sfs[l] * ys[l]
                    sb = sb + sfs[l + 1] * ys[l + 1]
                sdot = sa + sb
                for l in range(L):
                    m = wv * (sfs[l] + sdot * ys[l])
                    plsc.store_scatter(sf_v.at[d], [rows, cols[l]], m)

    for d in range(DEPTH):
        issue_lin(d, d)

    def it_body(i, carry):
        b0 = DEPTH * i
        for d in range(DEPTH):
            @pl.when(i > 0)
            def _(d=d):
                drain_scatter(d)

            drain_lin(d, b0 + d)
            copy_ridx_idx_issue_gather(d)
        for d in range(DEPTH):
            drain_gather(d)
            compute(d)
            issue_scatter(d)

            @pl.when(i < NI - 1)
            def _(d=d, b=b0 + d):
                issue_lin(d, b + DEPTH)

        return carry

    lax.fori_loop(0, NI, it_body, 0)
    for d in range(DEPTH):
        drain_scatter(d)
    plsc.subcore_barrier()
    pltpu.sync_copy(acc.at[pl.ds(s * TILE_N, TILE_N)],
                    out.at[pl.ds(cN + s * TILE_N, TILE_N)])


_sc_interact = functools.partial(
    pl.kernel,
    out_type=jax.ShapeDtypeStruct((2 * N_PAD, CW), jnp.float32),
    mesh=plsc.VectorSubcoreMesh(core_axis_name="c", subcore_axis_name="s"),
    scratch_types=[
        pltpu.VMEM((DEPTH, K), jnp.int32),
        pltpu.VMEM((DEPTH, K), jnp.int32),
        pltpu.VMEM((DEPTH, K), jnp.int32),
        pltpu.VMEM((DEPTH, K), jnp.int32),
        pltpu.VMEM((DEPTH, L, K), jnp.float32),
        pltpu.VMEM((DEPTH, FH, K), jnp.float32),
        pltpu.VMEM((DEPTH, K, CW), jnp.float32),
        pltpu.VMEM_SHARED((N_PAD, CW), jnp.float32),
        pltpu.SemaphoreType.DMA((DEPTH,)),
        pltpu.SemaphoreType.DMA((DEPTH,)),
        pltpu.SemaphoreType.DMA((DEPTH,)),
    ],
    compiler_params=pltpu.CompilerParams(use_tc_tiling_on_sc=False),
)(_sc_body)


def kernel(vectors, senders, receivers, node_species, varepsilon, embed,
           Wr1_0, Wr2_0, Wmix_0, Wprod_0, Wskipfirst_0,
           Wr1_1, Wr2_1, Wmix_1, Wprod_1, Wskip_1, Wm1, Wm2):
    f32 = jnp.float32
    pad_e = E_PAD - N_EDGES
    vt = jnp.concatenate(
        [vectors.astype(f32).T,
         jnp.concatenate([jnp.ones((1, pad_e), f32),
                          jnp.zeros((2, pad_e), f32)], axis=0)], axis=1)
    snd = jnp.concatenate(
        [senders.astype(jnp.int32), jnp.zeros((pad_e,), jnp.int32)])
    rcv = jnp.concatenate(
        [receivers.astype(jnp.int32),
         jnp.full((pad_e,), N_NODES, jnp.int32)])
    sp = jnp.concatenate(
        [node_species.astype(jnp.int32),
         jnp.zeros((N_PAD - N_NODES,), jnp.int32)])[:, None]
    eps11 = (1.0 / jnp.sqrt(1.0 + varepsilon ** 2)).reshape(1, 1).astype(f32)
    eye = jnp.eye(L, dtype=f32)
    t0 = jnp.zeros((S, F * L), f32).at[:, jnp.arange(F) * L].set(embed)
    msf = jnp.concatenate([jnp.kron(Wskipfirst_0[s2], eye) for s2 in range(S)],
                          axis=0)
    mskip = jnp.concatenate([jnp.kron(Wskip_1[s2], eye) for s2 in range(S)],
                            axis=0)
    mmix0 = jnp.kron(Wmix_0, eye)
    mmix1 = jnp.kron(Wmix_1, eye)
    wp0 = Wprod_0.reshape(S, F * 3)
    wp1 = Wprod_1.reshape(S, F * 3)
    k1 = jnp.zeros((F * L, H), f32).at[jnp.arange(F) * L].set(Wm1)
    zeros_tile = jnp.zeros((TILE_N, CW), f32)

    Yt, w0t, w1t = _edge_precompute(vt, eps11, Wr1_0.T, Wr2_0.T, Wr1_1.T,
                                    Wr2_1.T)

    rcv2d = rcv.reshape(E_PAD // KS, KS)
    f0 = _init_feats(sp, t0)
    a0 = _sc_interact(f0.reshape(2 * N_PAD, CW), w0t, snd, rcv2d, Yt,
                      zeros_tile)
    f1 = _node1(a0.reshape(2, N_PAD, CW), sp, msf, wp0, mmix0)
    a1 = _sc_interact(f1.reshape(2 * N_PAD, CW), w1t, snd, rcv2d, Yt,
                      zeros_tile)
    out = _node2(a1.reshape(2, N_PAD, CW), f1, sp, mskip, wp1, mmix1, k1, Wm2)
    return out[:N_NODES]
